# Initial kernel scaffold; baseline (speedup 1.0000x reference)
import jax
import jax.numpy as jnp
from jax.experimental import pallas as pl


def kernel(x, kv_cache, Wg, bg, Wa, ba, W1, b1, W2, b2):
    raise NotImplementedError("write your pallas kernel here")


# speedup vs baseline: 2.1586x; 2.1586x over previous
import functools

import jax
import jax.numpy as jnp
from jax.experimental import pallas as pl
from jax.experimental.pallas import tpu as pltpu

E = 8
K = 2
D = 768
H = 1536
S = 2048
T = 256


def _moe_body(x_ref, wg_ref, bg_ref, wa_ref, ba_ref, w1_ref, b1_ref,
              w2_ref, b2_ref, out_ref, acc_ref):
    e = pl.program_id(0)
    x = x_ref[...]

    logits = jnp.dot(x, wg_ref[...], preferred_element_type=jnp.float32)
    logits = logits + bg_ref[...]
    eidx = jax.lax.broadcasted_iota(jnp.int32, logits.shape, 1)
    m0 = jnp.max(logits, axis=1, keepdims=True)
    a0 = jnp.argmax(logits, axis=1).reshape(-1, 1)
    masked = jnp.where(eidx == a0, -jnp.inf, logits)
    m1 = jnp.max(masked, axis=1, keepdims=True)
    a1 = jnp.argmax(masked, axis=1).reshape(-1, 1)
    w0 = 1.0 / (1.0 + jnp.exp(m1 - m0))
    w1 = 1.0 - w0
    cw = w0 * (a0 == e).astype(jnp.float32) + w1 * (a1 == e).astype(jnp.float32)

    a = jnp.dot(x, wa_ref[0], preferred_element_type=jnp.float32) + ba_ref[0]
    a = a * jax.nn.sigmoid(a)
    f1 = jnp.dot(x, w1_ref[0], preferred_element_type=jnp.float32) + b1_ref[0]
    h = a * f1
    o = jnp.dot(h, w2_ref[0], preferred_element_type=jnp.float32) + b2_ref[0]

    t = pl.program_id(1)
    sl = pl.ds(t * T, T)

    @pl.when(e == 0)
    def _init():
        acc_ref[sl, :] = cw * o

    @pl.when(e != 0)
    def _accum():
        acc_ref[sl, :] += cw * o

    @pl.when(e == E - 1)
    def _fin():
        out_ref[...] = acc_ref[sl, :]


@functools.partial(jax.jit)
def _moe(x2, Wg, bg, Wa, ba, W1, b1, W2, b2):
    grid = (E, S // T)
    out = pl.pallas_call(
        _moe_body,
        grid=grid,
        in_specs=[
            pl.BlockSpec((T, D), lambda e, t: (t, 0)),
            pl.BlockSpec((D, E), lambda e, t: (0, 0)),
            pl.BlockSpec((E,), lambda e, t: (0,)),
            pl.BlockSpec((1, D, H), lambda e, t: (e, 0, 0)),
            pl.BlockSpec((1, 1, H), lambda e, t: (e, 0, 0)),
            pl.BlockSpec((1, D, H), lambda e, t: (e, 0, 0)),
            pl.BlockSpec((1, 1, H), lambda e, t: (e, 0, 0)),
            pl.BlockSpec((1, H, D), lambda e, t: (e, 0, 0)),
            pl.BlockSpec((1, 1, D), lambda e, t: (e, 0, 0)),
        ],
        out_specs=pl.BlockSpec((T, D), lambda e, t: (t, 0)),
        out_shape=jax.ShapeDtypeStruct((S, D), jnp.float32),
        scratch_shapes=[pltpu.VMEM((S, D), jnp.float32)],
        compiler_params=pltpu.CompilerParams(
            dimension_semantics=("arbitrary", "arbitrary"),
        ),
    )(x2, Wg, bg, Wa, ba.reshape(E, 1, H), W1, b1.reshape(E, 1, H),
      W2, b2.reshape(E, 1, D))
    return out


def kernel(x, kv_cache, Wg, bg, Wa, ba, W1, b1, W2, b2):
    B = x.shape[0]
    x2 = x.reshape(S, D)
    out = _moe(x2, Wg, bg, Wa, ba, W1, b1, W2, b2)
    return out.astype(jnp.float16).reshape(B, S, D)


# speedup vs baseline: 3.0565x; 1.4160x over previous
import functools

import jax
import jax.numpy as jnp
from jax.experimental import pallas as pl
from jax.experimental.pallas import tpu as pltpu
from jax.experimental.pallas import tpu_sc as plsc

E = 8
K = 2
D = 768
H = 1536
S = 2048
T = 256
NT = 24
P = NT * T
CHUNK = 256


def _routing_body(x_ref, wg_ref, bg_ref, pos0_ref, pos1_ref, w0_ref, w1_ref,
                  te_ref, act_ref):
    x = x_ref[...]
    logits = jnp.dot(x, wg_ref[...], preferred_element_type=jnp.float32)
    logits = logits + bg_ref[...]
    eidx = jax.lax.broadcasted_iota(jnp.int32, (S, E), 1)
    m0 = jnp.max(logits, axis=1, keepdims=True)
    a0 = jnp.argmax(logits, axis=1).reshape(-1, 1)
    masked = jnp.where(eidx == a0, -jnp.inf, logits)
    m1 = jnp.max(masked, axis=1, keepdims=True)
    a1 = jnp.argmax(masked, axis=1).reshape(-1, 1)
    w0 = 1.0 / (1.0 + jnp.exp(m1 - m0))
    w1 = 1.0 - w0
    w0_ref[...] = jnp.broadcast_to(w0, (S, 128))
    w1_ref[...] = jnp.broadcast_to(w1, (S, 128))

    oh0 = (eidx == a0).astype(jnp.float32)
    oh1 = (eidx == a1).astype(jnp.float32)
    ohsum = oh0 + oh1

    r_i = jax.lax.broadcasted_iota(jnp.int32, (CHUNK, CHUNK), 0)
    c_i = jax.lax.broadcasted_iota(jnp.int32, (CHUNK, CHUNK), 1)
    Lt = (c_i < r_i).astype(jnp.float32)
    carry = jnp.zeros((1, E), jnp.float32)
    excs = []
    for i in range(S // CHUNK):
        blk = ohsum[i * CHUNK:(i + 1) * CHUNK, :]
        excs.append(jnp.dot(Lt, blk, preferred_element_type=jnp.float32) + carry)
        carry = carry + jnp.sum(blk, axis=0, keepdims=True)
    exc = jnp.concatenate(excs, axis=0)
    counts = carry

    ci = counts.astype(jnp.int32)
    pc = ((ci + (T - 1)) // T) * T
    e_r = jax.lax.broadcasted_iota(jnp.int32, (E, E), 0)
    e_c = jax.lax.broadcasted_iota(jnp.int32, (E, E), 1)
    base = jnp.sum(jnp.where(e_c < e_r, jnp.broadcast_to(pc, (E, E)), 0),
                   axis=1).reshape(1, E)
    cc = base + pc

    basef = base.astype(jnp.float32)
    pos0 = jnp.sum(oh0 * (basef + exc), axis=1, keepdims=True)
    pos1 = jnp.sum(oh1 * (basef + exc), axis=1, keepdims=True)
    pos0_ref[...] = pos0.astype(jnp.int32)
    pos1_ref[...] = pos1.astype(jnp.int32)

    t_i = jax.lax.broadcasted_iota(jnp.int32, (NT, E), 0) * T
    te = jnp.sum((t_i >= jnp.broadcast_to(cc, (NT, E))).astype(jnp.int32),
                 axis=1, keepdims=True)
    act_ref[...] = (te < E).astype(jnp.int32)
    te_ref[...] = jnp.minimum(te, E - 1)


@functools.partial(jax.jit)
def _routing(x2, Wg, bg):
    return pl.pallas_call(
        _routing_body,
        out_shape=[
            jax.ShapeDtypeStruct((S, 1), jnp.int32),
            jax.ShapeDtypeStruct((S, 1), jnp.int32),
            jax.ShapeDtypeStruct((S, 128), jnp.float32),
            jax.ShapeDtypeStruct((S, 128), jnp.float32),
            jax.ShapeDtypeStruct((NT, 1), jnp.int32),
            jax.ShapeDtypeStruct((NT, 1), jnp.int32),
        ],
    )(x2, Wg, bg)


def _ffn_body(te_ref, act_ref, xs_ref, rw_ref, wa_ref, ba_ref, w1_ref, b1_ref,
              w2_ref, b2_ref, y_ref):
    t = pl.program_id(0)

    @pl.when(act_ref[t] == 1)
    def _go():
        x = xs_ref[...]
        a = jnp.dot(x, wa_ref[0], preferred_element_type=jnp.float32) + ba_ref[0]
        a = a * jax.nn.sigmoid(a)
        f1 = jnp.dot(x, w1_ref[0], preferred_element_type=jnp.float32) + b1_ref[0]
        h = a * f1
        o = jnp.dot(h, w2_ref[0], preferred_element_type=jnp.float32) + b2_ref[0]
        y_ref[...] = o * rw_ref[...][:, 0:1]

    @pl.when(act_ref[t] == 0)
    def _skip():
        y_ref[...] = jnp.zeros_like(y_ref)


@functools.partial(jax.jit)
def _ffn(xs, rw, te, act, Wa, ba, W1, b1, W2, b2):
    grid_spec = pltpu.PrefetchScalarGridSpec(
        num_scalar_prefetch=2,
        grid=(NT,),
        in_specs=[
            pl.BlockSpec((T, D), lambda t, te, act: (t, 0)),
            pl.BlockSpec((T, 128), lambda t, te, act: (t, 0)),
            pl.BlockSpec((1, D, H), lambda t, te, act: (te[t], 0, 0)),
            pl.BlockSpec((1, 1, H), lambda t, te, act: (te[t], 0, 0)),
            pl.BlockSpec((1, D, H), lambda t, te, act: (te[t], 0, 0)),
            pl.BlockSpec((1, 1, H), lambda t, te, act: (te[t], 0, 0)),
            pl.BlockSpec((1, H, D), lambda t, te, act: (te[t], 0, 0)),
            pl.BlockSpec((1, 1, D), lambda t, te, act: (te[t], 0, 0)),
        ],
        out_specs=pl.BlockSpec((T, D), lambda t, te, act: (t, 0)),
    )
    return pl.pallas_call(
        _ffn_body,
        grid_spec=grid_spec,
        out_shape=jax.ShapeDtypeStruct((P, D), jnp.float32),
        compiler_params=pltpu.CompilerParams(
            dimension_semantics=("arbitrary",),
        ),
    )(te, act, xs, rw, Wa, ba.reshape(E, 1, H), W1, b1.reshape(E, 1, H),
      W2, b2.reshape(E, 1, D))


_SC_NC = 2
_SC_NS = 16
_NW = _SC_NC * _SC_NS
_TPW = S // _NW


def _sc_mesh():
    return plsc.VectorSubcoreMesh(core_axis_name="c", subcore_axis_name="s")


@functools.partial(
    pl.kernel,
    out_type=[
        jax.ShapeDtypeStruct((P, D), jnp.float32),
        jax.ShapeDtypeStruct((P, 128), jnp.float32),
    ],
    mesh=_sc_mesh(),
    scratch_types=[
        pltpu.VMEM((_TPW,), jnp.int32),
        pltpu.VMEM((_TPW,), jnp.int32),
        pltpu.VMEM((_TPW, D), jnp.float32),
        pltpu.VMEM((_TPW, 128), jnp.float32),
        pltpu.VMEM((_TPW, 128), jnp.float32),
        pltpu.SemaphoreType.DMA,
        pltpu.SemaphoreType.DMA,
        pltpu.SemaphoreType.DMA,
        pltpu.SemaphoreType.DMA,
    ],
)
def _sc_dispatch(x_hbm, pos0_hbm, pos1_hbm, w0_hbm, w1_hbm, xs_hbm, rw_hbm,
                 idx0_v, idx1_v, rows_v, w0_v, w1_v, s0, s1, s2, s3):
    wid = jax.lax.axis_index("s") * _SC_NC + jax.lax.axis_index("c")
    base = wid * _TPW
    pltpu.sync_copy(pos0_hbm.at[wid], idx0_v)
    pltpu.sync_copy(pos1_hbm.at[wid], idx1_v)
    pltpu.sync_copy(x_hbm.at[pl.ds(base, _TPW)], rows_v)
    pltpu.sync_copy(w0_hbm.at[pl.ds(base, _TPW)], w0_v)
    pltpu.sync_copy(w1_hbm.at[pl.ds(base, _TPW)], w1_v)
    c0 = pltpu.async_copy(rows_v, xs_hbm.at[idx0_v], s0)
    c1 = pltpu.async_copy(rows_v, xs_hbm.at[idx1_v], s1)
    c2 = pltpu.async_copy(w0_v, rw_hbm.at[idx0_v], s2)
    c3 = pltpu.async_copy(w1_v, rw_hbm.at[idx1_v], s3)
    c0.wait()
    c1.wait()
    c2.wait()
    c3.wait()


@functools.partial(
    pl.kernel,
    out_type=jax.ShapeDtypeStruct((S, D), jnp.float32),
    mesh=_sc_mesh(),
    scratch_types=[
        pltpu.VMEM((_TPW,), jnp.int32),
        pltpu.VMEM((_TPW,), jnp.int32),
        pltpu.VMEM((_TPW, D), jnp.float32),
        pltpu.VMEM((_TPW, D), jnp.float32),
        pltpu.SemaphoreType.DMA,
        pltpu.SemaphoreType.DMA,
    ],
)
def _sc_combine(y_hbm, pos0_hbm, pos1_hbm, out_hbm,
                idx0_v, idx1_v, rows0_v, rows1_v, s0, s1):
    wid = jax.lax.axis_index("s") * _SC_NC + jax.lax.axis_index("c")
    base = wid * _TPW
    pltpu.sync_copy(pos0_hbm.at[wid], idx0_v)
    pltpu.sync_copy(pos1_hbm.at[wid], idx1_v)
    c0 = pltpu.async_copy(y_hbm.at[idx0_v], rows0_v, s0)
    c1 = pltpu.async_copy(y_hbm.at[idx1_v], rows1_v, s1)
    c0.wait()
    c1.wait()

    def body_i(i, carry):
        for j in range(D // 16):
            sl = pl.ds(j * 16, 16)
            rows0_v[i, sl] = rows0_v[i, sl] + rows1_v[i, sl]
        return carry

    jax.lax.fori_loop(0, _TPW, body_i, 0)
    pltpu.sync_copy(rows0_v, out_hbm.at[pl.ds(base, _TPW)])


def kernel(x, kv_cache, Wg, bg, Wa, ba, W1, b1, W2, b2):
    B = x.shape[0]
    x2 = x.reshape(S, D)
    pos0, pos1, w0, w1, te, act = _routing(x2, Wg, bg)
    pos0w = pos0.reshape(_NW, _TPW)
    pos1w = pos1.reshape(_NW, _TPW)
    xs, rw = _sc_dispatch(x2, pos0w, pos1w, w0, w1)
    y = _ffn(xs, rw, te.reshape(NT), act.reshape(NT), Wa, ba, W1, b1, W2, b2)
    out = _sc_combine(y, pos0w, pos1w)
    return out.astype(jnp.float16).reshape(B, S, D)
